# baseline (speedup 1.0000x reference)
import functools

import jax
import jax.numpy as jnp
from jax import lax
from jax.experimental import pallas as pl
from jax.experimental.pallas import tpu as pltpu
from jax.experimental.pallas import tpu_sc as plsc

_N = 10000
_E = 160000
_F = 128
_D = 128
_C = 10
_R = 4
_G = 64
_P = 0.1
_LAYERS = 4
_M = _R * _N
_EPS = 1e-5

_NW = 32
_NRANGE = 64
_RPT = _M // _NRANGE
_BLK = 128
_GRP = _BLK // 16

_BR = 2000
_NBLK = _M // _BR
_NXB = _N // _BR


def _make_agg_kernel():
    mesh = plsc.VectorSubcoreMesh(core_axis_name="c", subcore_axis_name="s",
                                  num_cores=2, num_subcores=16)
    nc = 2

    @functools.partial(
        pl.kernel,
        out_type=jax.ShapeDtypeStruct((_NRANGE, _RPT, _D), jnp.float32),
        mesh=mesh,
        compiler_params=pltpu.CompilerParams(needs_layout_passes=False),
        scratch_types=[
            pltpu.VMEM((_BLK,), jnp.int32),
            pltpu.VMEM((_BLK,), jnp.int32),
            pltpu.VMEM((_BLK, _D), jnp.float32),
            pltpu.VMEM((_BLK, _D), jnp.float32),
            pltpu.VMEM((_RPT, _D), jnp.float32),
            pltpu.SMEM((_BLK,), jnp.int32),
            pltpu.SMEM((_NRANGE + 8,), jnp.int32),
            pltpu.VMEM_SHARED((_NRANGE + 8,), jnp.int32),
            pltpu.VMEM_SHARED((32, _BLK), jnp.int32),
            pltpu.SemaphoreType.DMA,
            pltpu.SemaphoreType.DMA,
            pltpu.SemaphoreType.DMA,
            pltpu.SemaphoreType.DMA,
            pltpu.SemaphoreType.DMA,
            pltpu.SemaphoreType.DMA,
        ],
    )
    def agg_kernel(h_hbm, src_hbm, dst_hbm, bounds_hbm, agg_hbm,
                   idx_a, idx_b, rows_a, rows_b, acc_v, dst_s, bnd_s,
                   bnd_sh, spd, sem_ia, sem_ib, sem_ra, sem_rb,
                   sem_da, sem_db):
        cid = lax.axis_index("c")
        sid = lax.axis_index("s")
        wid = sid * nc + cid
        slot_a = sid * 2
        slot_b = sid * 2 + 1

        @pl.when(sid == 0)
        def _():
            pltpu.sync_copy(bounds_hbm, bnd_sh)

        plsc.subcore_barrier()
        pltpu.sync_copy(bnd_sh, bnd_s)

        zero16 = jnp.zeros((16,), jnp.float32)

        def stage_dst(base, slot, sem_d):
            drow = lax.shift_right_logical(base, 7)
            pltpu.make_async_copy(dst_hbm.at[drow], spd.at[slot],
                                  sem_d).wait()
            pltpu.sync_copy(spd.at[slot], dst_s)

        def edge_loop(base, rows_v, lo, hi, row_base):
            jlo = jnp.where(lo > base, lo - base, 0)
            jhi = jnp.where(hi - base < _BLK, hi - base, _BLK)
            n = jhi - jlo

            def one(j):
                r = dst_s[j] - row_base
                for c in range(_D // 16):
                    plsc.addupdate(acc_v.at[r, pl.ds(c * 16, 16)],
                                   rows_v[j, pl.ds(c * 16, 16)])

            @pl.loop(0, lax.shift_right_logical(n, 2))
            def _edge4(k):
                j = jlo + lax.shift_left(k, 2)
                one(j)
                one(j + 1)
                one(j + 2)
                one(j + 3)

            @pl.loop(jlo + jnp.bitwise_and(n, -4), jhi)
            def _edge(j):
                one(j)

        @pl.loop(0, 2)
        def _range(qi):
            q = wid * 2 + qi
            row_base = q * _RPT
            lo = bnd_s[q]
            hi = bnd_s[q + 1]

            @pl.loop(0, _RPT * (_D // 16))
            def _zero(i):
                r = lax.shift_right_logical(i, 3)
                c = lax.shift_left(jnp.bitwise_and(i, 7), 4)
                acc_v[r, pl.ds(c, 16)] = zero16

            lo_al = lax.shift_left(lax.shift_right_logical(lo, 7), 7)
            nblk = lax.shift_right_logical(hi - lo_al + _BLK - 1, 7)

            def src_copy(blkno, idx_v, sem, slot, sem_d):
                base = pl.multiple_of(lo_al + lax.shift_left(blkno, 7), _BLK)
                pltpu.async_copy(dst_hbm.at[lax.shift_right_logical(base, 7)],
                                 spd.at[slot], sem_d)
                return pltpu.async_copy(src_hbm.at[pl.ds(base, _BLK)],
                                        idx_v, sem)

            @pl.when(nblk > 0)
            def _():
                src_copy(0, idx_a, sem_ia, slot_a, sem_da).wait()
                pltpu.async_copy(h_hbm.at[idx_a], rows_a, sem_ra)

                @pl.when(nblk > 1)
                def _():
                    src_copy(1, idx_b, sem_ib, slot_b, sem_db)

            @pl.when(nblk > 1)
            def _():
                pltpu.make_async_copy(
                    src_hbm.at[pl.ds(pl.multiple_of(lo_al + _BLK, _BLK),
                                     _BLK)],
                    idx_b, sem_ib).wait()
                pltpu.async_copy(h_hbm.at[idx_b], rows_b, sem_rb)

            def half_step(blk, idx_v, sem_i, rows_v, sem_r, slot, sem_d,
                          base):
                pltpu.make_async_copy(h_hbm.at[idx_v], rows_v, sem_r).wait()
                stage_dst(base, slot, sem_d)

                @pl.when(blk + 2 < nblk)
                def _():
                    src_copy(blk + 2, idx_v, sem_i, slot, sem_d)

                edge_loop(base, rows_v, lo, hi, row_base)

                @pl.when(blk + 2 < nblk)
                def _():
                    pltpu.make_async_copy(
                        src_hbm.at[pl.ds(pl.multiple_of(base + 2 * _BLK,
                                                        _BLK), _BLK)],
                        idx_v, sem_i).wait()
                    pltpu.async_copy(h_hbm.at[idx_v], rows_v, sem_r)

            @pl.loop(0, lax.shift_right_logical(nblk + 1, 1))
            def _pair(i):
                a = lax.shift_left(i, 1)
                b = a + 1
                basea = pl.multiple_of(lo_al + lax.shift_left(a, 7), _BLK)
                baseb = basea + _BLK

                half_step(a, idx_a, sem_ia, rows_a, sem_ra, slot_a, sem_da,
                          basea)

                @pl.when(b < nblk)
                def _():
                    half_step(b, idx_b, sem_ib, rows_b, sem_rb, slot_b,
                              sem_db, baseb)

            pltpu.sync_copy(acc_v, agg_hbm.at[q])

    return agg_kernel


_AGG_CACHE = []


def _agg(h, src_flat, dst_flat, bounds):
    if not _AGG_CACHE:
        _AGG_CACHE.append(_make_agg_kernel())
    return _AGG_CACHE[0](h, src_flat, dst_flat, bounds)


def _h0_body(x_ref, keep_ref, h0_ref):
    h0_ref[...] = x_ref[...] * keep_ref[...]


def _h0(x, keep):
    return pl.pallas_call(
        _h0_body,
        grid=(_NBLK,),
        in_specs=[
            pl.BlockSpec((_BR, _F), lambda i: (i % _NXB, 0)),
            pl.BlockSpec((_BR, 1), lambda i: (i, 0)),
        ],
        out_specs=pl.BlockSpec((_BR, _F), lambda i: (i, 0)),
        out_shape=jax.ShapeDtypeStruct((_M, _F), jnp.float32),
    )(x, keep)


def _phase_a_body(h_ref, agg_ref, w1_ref, batch_ref, y1_ref, st_ref, pool_ref):
    @pl.when(pl.program_id(0) == 0)
    def _():
        st_ref[...] = jnp.zeros_like(st_ref)
        pool_ref[...] = jnp.zeros_like(pool_ref)

    h = h_ref[...]
    z = h + agg_ref[...]
    y1 = jnp.dot(z, w1_ref[...], preferred_element_type=jnp.float32)
    y1_ref[...] = y1
    s0 = jnp.sum(y1, axis=0, keepdims=True)
    s1 = jnp.sum(y1 * y1, axis=0, keepdims=True)
    st_ref[...] += jnp.concatenate(
        [s0, s1, jnp.zeros((6, _D), jnp.float32)], axis=0)
    b = batch_ref[0, 0, :]
    oh = (b[None, :] == lax.broadcasted_iota(jnp.int32, (_G, _BR), 0)
          ).astype(jnp.float32)
    pool_ref[...] += jnp.dot(oh, h, preferred_element_type=jnp.float32)


def _phase_a(h, agg, w1, batch3):
    return pl.pallas_call(
        _phase_a_body,
        grid=(_NBLK,),
        in_specs=[
            pl.BlockSpec((_BR, _D), lambda i: (i, 0)),
            pl.BlockSpec((_BR, _D), lambda i: (i, 0)),
            pl.BlockSpec((_D, _D), lambda i: (0, 0)),
            pl.BlockSpec((1, 1, _BR), lambda i: (i % _NXB, 0, 0)),
        ],
        out_specs=[
            pl.BlockSpec((_BR, _D), lambda i: (i, 0)),
            pl.BlockSpec((8, _D), lambda i: (0, 0)),
            pl.BlockSpec((_G, _D), lambda i: (0, 0)),
        ],
        out_shape=[
            jax.ShapeDtypeStruct((_M, _D), jnp.float32),
            jax.ShapeDtypeStruct((8, _D), jnp.float32),
            jax.ShapeDtypeStruct((_G, _D), jnp.float32),
        ],
    )(h, agg, w1, batch3)


def _bn_from_stats(y, st_ref, g, b):
    mean = st_ref[0:1, :] * (1.0 / _M)
    var = st_ref[1:2, :] * (1.0 / _M) - mean * mean
    inv = lax.rsqrt(var + _EPS)
    return (y - mean) * inv * g + b


def _phase_b_body(y1_ref, st_ref, w2_ref, g1_ref, b1_ref, y2_ref, st2_ref):
    @pl.when(pl.program_id(0) == 0)
    def _():
        st2_ref[...] = jnp.zeros_like(st2_ref)

    y1n = _bn_from_stats(y1_ref[...], st_ref, g1_ref[...], b1_ref[...])
    y1n = jnp.maximum(y1n, 0.0)
    y2 = jnp.dot(y1n, w2_ref[...], preferred_element_type=jnp.float32)
    y2_ref[...] = y2
    s0 = jnp.sum(y2, axis=0, keepdims=True)
    s1 = jnp.sum(y2 * y2, axis=0, keepdims=True)
    st2_ref[...] += jnp.concatenate(
        [s0, s1, jnp.zeros((6, _D), jnp.float32)], axis=0)


def _phase_b(y1, st1, w2, g1, b1):
    return pl.pallas_call(
        _phase_b_body,
        grid=(_NBLK,),
        in_specs=[
            pl.BlockSpec((_BR, _D), lambda i: (i, 0)),
            pl.BlockSpec((8, _D), lambda i: (0, 0)),
            pl.BlockSpec((_D, _D), lambda i: (0, 0)),
            pl.BlockSpec((1, _D), lambda i: (0, 0)),
            pl.BlockSpec((1, _D), lambda i: (0, 0)),
        ],
        out_specs=[
            pl.BlockSpec((_BR, _D), lambda i: (i, 0)),
            pl.BlockSpec((8, _D), lambda i: (0, 0)),
        ],
        out_shape=[
            jax.ShapeDtypeStruct((_M, _D), jnp.float32),
            jax.ShapeDtypeStruct((8, _D), jnp.float32),
        ],
    )(y1, st1, w2, g1, b1)


def _phase_c_body(y2_ref, st_ref, g2_ref, b2_ref, h_ref):
    h = _bn_from_stats(y2_ref[...], st_ref, g2_ref[...], b2_ref[...])
    h_ref[...] = jnp.maximum(h, 0.0)


def _phase_c(y2, st2, g2, b2):
    return pl.pallas_call(
        _phase_c_body,
        grid=(_NBLK,),
        in_specs=[
            pl.BlockSpec((_BR, _D), lambda i: (i, 0)),
            pl.BlockSpec((8, _D), lambda i: (0, 0)),
            pl.BlockSpec((1, _D), lambda i: (0, 0)),
            pl.BlockSpec((1, _D), lambda i: (0, 0)),
        ],
        out_specs=pl.BlockSpec((_BR, _D), lambda i: (i, 0)),
        out_shape=jax.ShapeDtypeStruct((_M, _D), jnp.float32),
    )(y2, st2, g2, b2)


def _pool_body(h_ref, batch_ref, pool_ref):
    @pl.when(pl.program_id(0) == 0)
    def _():
        pool_ref[...] = jnp.zeros_like(pool_ref)

    b = batch_ref[0, 0, :]
    oh = (b[None, :] == lax.broadcasted_iota(jnp.int32, (_G, _BR), 0)
          ).astype(jnp.float32)
    pool_ref[...] += jnp.dot(oh, h_ref[...],
                             preferred_element_type=jnp.float32)


def _pool(h, batch3):
    return pl.pallas_call(
        _pool_body,
        grid=(_NBLK,),
        in_specs=[
            pl.BlockSpec((_BR, _D), lambda i: (i, 0)),
            pl.BlockSpec((1, 1, _BR), lambda i: (i % _NXB, 0, 0)),
        ],
        out_specs=pl.BlockSpec((_G, _D), lambda i: (0, 0)),
        out_shape=jax.ShapeDtypeStruct((_G, _D), jnp.float32),
    )(h, batch3)


def _head_body(pools_ref, fcw_ref, fcb_ref, out_ref):
    acc = jnp.zeros((_G, _C), jnp.float32)
    for i in range(_LAYERS + 1):
        acc += jnp.dot(pools_ref[i] * (1.0 / _R), fcw_ref[i],
                       preferred_element_type=jnp.float32)
    acc += jnp.sum(fcb_ref[...], axis=0, keepdims=True)
    mx = jnp.max(acc, axis=-1, keepdims=True)
    sh = acc - mx
    out_ref[...] = sh - jnp.log(jnp.sum(jnp.exp(sh), axis=-1, keepdims=True))


def _head(pools, fc_w, fc_b):
    return pl.pallas_call(
        _head_body,
        out_shape=jax.ShapeDtypeStruct((_G, _C), jnp.float32),
    )(pools, fc_w, fc_b)


def kernel(x, edge_index, batch, conv_w1, conv_b1, conv_bng, conv_bnb,
           conv_w2, conv_b2, bn_g, bn_b, fc_w, fc_b):
    del conv_b1, conv_b2

    drop = jax.random.bernoulli(jax.random.key(42), _P, (_R, _N))
    keep = jnp.where(drop, 0.0, 1.0).astype(jnp.float32).reshape(_M, 1)

    offset = (jnp.max(edge_index) + 1).astype(jnp.int32)
    src, dst = edge_index[0], edge_index[1]
    perm = jnp.argsort(dst)
    dst_s = dst[perm]
    src_s = src[perm]
    roff = offset * jnp.arange(_R, dtype=jnp.int32)
    src_flat = (src_s[None, :] + roff[:, None]).reshape(_R * _E)
    dst_flat = (dst_s[None, :] + roff[:, None]).reshape(_R * _E)
    npad = 5120 * _BLK - _R * _E
    pad = jnp.full((npad,), 0, jnp.int32)
    padd = jnp.full((npad,), jnp.int32(1 << 29), jnp.int32)
    src_flat = jnp.concatenate([src_flat, pad])
    dst_flat = jnp.concatenate([dst_flat, padd])
    starts = jnp.arange(_NRANGE + 1, dtype=jnp.int32) * _RPT
    bounds = jnp.searchsorted(dst_flat[:_R * _E], starts).astype(jnp.int32)
    bounds = jnp.concatenate(
        [bounds, jnp.zeros((7,), jnp.int32)])

    batch3 = batch.reshape(_NXB, 1, _BR)

    h = _h0(x, keep)
    pools = [None] * (_LAYERS + 1)
    for i in range(_LAYERS):
        agg = _agg(h, src_flat, dst_flat.reshape(5120, _BLK),
                   bounds).reshape(_M, _D)
        y1, st1, pools[i] = _phase_a(h, agg, conv_w1[i], batch3)
        y2, st2 = _phase_b(y1, st1, conv_w2[i], conv_bng[i][None, :],
                           conv_bnb[i][None, :])
        h = _phase_c(y2, st2, bn_g[i][None, :], bn_b[i][None, :])
    pools[_LAYERS] = _pool(h, batch3)

    return _head(jnp.stack(pools), fc_w, fc_b)


# speedup vs baseline: 1.0287x; 1.0287x over previous
import functools

import jax
import jax.numpy as jnp
from jax import lax
from jax.experimental import pallas as pl
from jax.experimental.pallas import tpu as pltpu
from jax.experimental.pallas import tpu_sc as plsc

_N = 10000
_E = 160000
_F = 128
_D = 128
_C = 10
_R = 4
_G = 64
_P = 0.1
_LAYERS = 4
_M = _R * _N
_EPS = 1e-5

_NW = 32
_NRANGE = 64
_RPT = _M // _NRANGE
_BLK = 128
_GRP = _BLK // 16

_BR = 2000
_NBLK = _M // _BR
_NXB = _N // _BR


def _make_agg_kernel():
    mesh = plsc.VectorSubcoreMesh(core_axis_name="c", subcore_axis_name="s",
                                  num_cores=2, num_subcores=16)
    nc = 2

    @functools.partial(
        pl.kernel,
        out_type=jax.ShapeDtypeStruct((_NRANGE, _RPT, _D), jnp.float32),
        mesh=mesh,
        compiler_params=pltpu.CompilerParams(needs_layout_passes=False),
        scratch_types=[
            pltpu.VMEM((_BLK,), jnp.int32),
            pltpu.VMEM((_BLK,), jnp.int32),
            pltpu.VMEM((_BLK, _D), jnp.float32),
            pltpu.VMEM((_BLK, _D), jnp.float32),
            pltpu.VMEM((_RPT, _D), jnp.float32),
            pltpu.SMEM((_BLK,), jnp.int32),
            pltpu.SMEM((_NRANGE + 8,), jnp.int32),
            pltpu.VMEM_SHARED((_NRANGE + 8,), jnp.int32),
            pltpu.VMEM_SHARED((32, _BLK), jnp.int32),
            pltpu.SemaphoreType.DMA,
            pltpu.SemaphoreType.DMA,
            pltpu.SemaphoreType.DMA,
            pltpu.SemaphoreType.DMA,
            pltpu.SemaphoreType.DMA,
            pltpu.SemaphoreType.DMA,
        ],
    )
    def agg_kernel(h_hbm, src_hbm, dst_hbm, bounds_hbm, agg_hbm,
                   idx_a, idx_b, rows_a, rows_b, acc_v, dst_s, bnd_s,
                   bnd_sh, spd, sem_ia, sem_ib, sem_ra, sem_rb,
                   sem_da, sem_db):
        cid = lax.axis_index("c")
        sid = lax.axis_index("s")
        wid = sid * nc + cid
        slot_a = sid * 2
        slot_b = sid * 2 + 1

        @pl.when(sid == 0)
        def _():
            pltpu.sync_copy(bounds_hbm, bnd_sh)

        plsc.subcore_barrier()
        pltpu.sync_copy(bnd_sh, bnd_s)

        zero16 = jnp.zeros((16,), jnp.float32)
        nq = 4
        qsz = _BLK // nq

        def gather_issue(idx_v, rows_v, sem_r):
            for k in range(nq):
                pltpu.async_copy(h_hbm.at[idx_v.at[pl.ds(k * qsz, qsz)]],
                                 rows_v.at[pl.ds(k * qsz, qsz), :], sem_r)

        def gather_wait(idx_v, rows_v, sem_r):
            for k in range(nq):
                pltpu.make_async_copy(
                    h_hbm.at[idx_v.at[pl.ds(k * qsz, qsz)]],
                    rows_v.at[pl.ds(k * qsz, qsz), :], sem_r).wait()

        def stage_dst(base, slot, sem_d):
            drow = lax.shift_right_logical(base, 7)
            pltpu.make_async_copy(dst_hbm.at[drow], spd.at[slot],
                                  sem_d).wait()
            pltpu.sync_copy(spd.at[slot], dst_s)

        def edge_loop(base, rows_v, lo, hi, row_base):
            jlo = jnp.where(lo > base, lo - base, 0)
            jhi = jnp.where(hi - base < _BLK, hi - base, _BLK)
            n = jhi - jlo

            del n

            @pl.loop(jlo, jhi)
            def _edge(j):
                r = dst_s[j] - row_base
                for c in range(_D // 16):
                    plsc.addupdate(acc_v.at[r, pl.ds(c * 16, 16)],
                                   rows_v[j, pl.ds(c * 16, 16)])

        @pl.loop(0, 2)
        def _range(qi):
            q = wid * 2 + qi
            row_base = q * _RPT
            lo = bnd_s[q]
            hi = bnd_s[q + 1]

            @pl.loop(0, _RPT * (_D // 16))
            def _zero(i):
                r = lax.shift_right_logical(i, 3)
                c = lax.shift_left(jnp.bitwise_and(i, 7), 4)
                acc_v[r, pl.ds(c, 16)] = zero16

            lo_al = lax.shift_left(lax.shift_right_logical(lo, 7), 7)
            nblk = lax.shift_right_logical(hi - lo_al + _BLK - 1, 7)

            def src_copy(blkno, idx_v, sem, slot, sem_d):
                base = pl.multiple_of(lo_al + lax.shift_left(blkno, 7), _BLK)
                pltpu.async_copy(dst_hbm.at[lax.shift_right_logical(base, 7)],
                                 spd.at[slot], sem_d)
                return pltpu.async_copy(src_hbm.at[pl.ds(base, _BLK)],
                                        idx_v, sem)

            @pl.when(nblk > 0)
            def _():
                src_copy(0, idx_a, sem_ia, slot_a, sem_da).wait()
                gather_issue(idx_a, rows_a, sem_ra)

                @pl.when(nblk > 1)
                def _():
                    src_copy(1, idx_b, sem_ib, slot_b, sem_db)

            @pl.when(nblk > 1)
            def _():
                pltpu.make_async_copy(
                    src_hbm.at[pl.ds(pl.multiple_of(lo_al + _BLK, _BLK),
                                     _BLK)],
                    idx_b, sem_ib).wait()
                gather_issue(idx_b, rows_b, sem_rb)

            def half_step(blk, idx_v, sem_i, rows_v, sem_r, slot, sem_d,
                          base):
                gather_wait(idx_v, rows_v, sem_r)
                stage_dst(base, slot, sem_d)

                @pl.when(blk + 2 < nblk)
                def _():
                    src_copy(blk + 2, idx_v, sem_i, slot, sem_d)

                edge_loop(base, rows_v, lo, hi, row_base)

                @pl.when(blk + 2 < nblk)
                def _():
                    pltpu.make_async_copy(
                        src_hbm.at[pl.ds(pl.multiple_of(base + 2 * _BLK,
                                                        _BLK), _BLK)],
                        idx_v, sem_i).wait()
                    gather_issue(idx_v, rows_v, sem_r)

            @pl.loop(0, lax.shift_right_logical(nblk + 1, 1))
            def _pair(i):
                a = lax.shift_left(i, 1)
                b = a + 1
                basea = pl.multiple_of(lo_al + lax.shift_left(a, 7), _BLK)
                baseb = basea + _BLK

                half_step(a, idx_a, sem_ia, rows_a, sem_ra, slot_a, sem_da,
                          basea)

                @pl.when(b < nblk)
                def _():
                    half_step(b, idx_b, sem_ib, rows_b, sem_rb, slot_b,
                              sem_db, baseb)

            pltpu.sync_copy(acc_v, agg_hbm.at[q])

    return agg_kernel


_AGG_CACHE = []


def _agg(h, src_flat, dst_flat, bounds):
    if not _AGG_CACHE:
        _AGG_CACHE.append(_make_agg_kernel())
    return _AGG_CACHE[0](h, src_flat, dst_flat, bounds)


def _h0_body(x_ref, keep_ref, h0_ref):
    h0_ref[...] = x_ref[...] * keep_ref[...]


def _h0(x, keep):
    return pl.pallas_call(
        _h0_body,
        grid=(_NBLK,),
        in_specs=[
            pl.BlockSpec((_BR, _F), lambda i: (i % _NXB, 0)),
            pl.BlockSpec((_BR, 1), lambda i: (i, 0)),
        ],
        out_specs=pl.BlockSpec((_BR, _F), lambda i: (i, 0)),
        out_shape=jax.ShapeDtypeStruct((_M, _F), jnp.float32),
    )(x, keep)


def _phase_a_body(h_ref, agg_ref, w1_ref, batch_ref, y1_ref, st_ref, pool_ref):
    @pl.when(pl.program_id(0) == 0)
    def _():
        st_ref[...] = jnp.zeros_like(st_ref)
        pool_ref[...] = jnp.zeros_like(pool_ref)

    h = h_ref[...]
    z = h + agg_ref[...]
    y1 = jnp.dot(z, w1_ref[...], preferred_element_type=jnp.float32)
    y1_ref[...] = y1
    s0 = jnp.sum(y1, axis=0, keepdims=True)
    s1 = jnp.sum(y1 * y1, axis=0, keepdims=True)
    st_ref[...] += jnp.concatenate(
        [s0, s1, jnp.zeros((6, _D), jnp.float32)], axis=0)
    b = batch_ref[0, 0, :]
    oh = (b[None, :] == lax.broadcasted_iota(jnp.int32, (_G, _BR), 0)
          ).astype(jnp.float32)
    pool_ref[...] += jnp.dot(oh, h, preferred_element_type=jnp.float32)


def _phase_a(h, agg, w1, batch3):
    return pl.pallas_call(
        _phase_a_body,
        grid=(_NBLK,),
        in_specs=[
            pl.BlockSpec((_BR, _D), lambda i: (i, 0)),
            pl.BlockSpec((_BR, _D), lambda i: (i, 0)),
            pl.BlockSpec((_D, _D), lambda i: (0, 0)),
            pl.BlockSpec((1, 1, _BR), lambda i: (i % _NXB, 0, 0)),
        ],
        out_specs=[
            pl.BlockSpec((_BR, _D), lambda i: (i, 0)),
            pl.BlockSpec((8, _D), lambda i: (0, 0)),
            pl.BlockSpec((_G, _D), lambda i: (0, 0)),
        ],
        out_shape=[
            jax.ShapeDtypeStruct((_M, _D), jnp.float32),
            jax.ShapeDtypeStruct((8, _D), jnp.float32),
            jax.ShapeDtypeStruct((_G, _D), jnp.float32),
        ],
    )(h, agg, w1, batch3)


def _bn_from_stats(y, st_ref, g, b):
    mean = st_ref[0:1, :] * (1.0 / _M)
    var = st_ref[1:2, :] * (1.0 / _M) - mean * mean
    inv = lax.rsqrt(var + _EPS)
    return (y - mean) * inv * g + b


def _phase_b_body(y1_ref, st_ref, w2_ref, g1_ref, b1_ref, y2_ref, st2_ref):
    @pl.when(pl.program_id(0) == 0)
    def _():
        st2_ref[...] = jnp.zeros_like(st2_ref)

    y1n = _bn_from_stats(y1_ref[...], st_ref, g1_ref[...], b1_ref[...])
    y1n = jnp.maximum(y1n, 0.0)
    y2 = jnp.dot(y1n, w2_ref[...], preferred_element_type=jnp.float32)
    y2_ref[...] = y2
    s0 = jnp.sum(y2, axis=0, keepdims=True)
    s1 = jnp.sum(y2 * y2, axis=0, keepdims=True)
    st2_ref[...] += jnp.concatenate(
        [s0, s1, jnp.zeros((6, _D), jnp.float32)], axis=0)


def _phase_b(y1, st1, w2, g1, b1):
    return pl.pallas_call(
        _phase_b_body,
        grid=(_NBLK,),
        in_specs=[
            pl.BlockSpec((_BR, _D), lambda i: (i, 0)),
            pl.BlockSpec((8, _D), lambda i: (0, 0)),
            pl.BlockSpec((_D, _D), lambda i: (0, 0)),
            pl.BlockSpec((1, _D), lambda i: (0, 0)),
            pl.BlockSpec((1, _D), lambda i: (0, 0)),
        ],
        out_specs=[
            pl.BlockSpec((_BR, _D), lambda i: (i, 0)),
            pl.BlockSpec((8, _D), lambda i: (0, 0)),
        ],
        out_shape=[
            jax.ShapeDtypeStruct((_M, _D), jnp.float32),
            jax.ShapeDtypeStruct((8, _D), jnp.float32),
        ],
    )(y1, st1, w2, g1, b1)


def _phase_c_body(y2_ref, st_ref, g2_ref, b2_ref, h_ref):
    h = _bn_from_stats(y2_ref[...], st_ref, g2_ref[...], b2_ref[...])
    h_ref[...] = jnp.maximum(h, 0.0)


def _phase_c(y2, st2, g2, b2):
    return pl.pallas_call(
        _phase_c_body,
        grid=(_NBLK,),
        in_specs=[
            pl.BlockSpec((_BR, _D), lambda i: (i, 0)),
            pl.BlockSpec((8, _D), lambda i: (0, 0)),
            pl.BlockSpec((1, _D), lambda i: (0, 0)),
            pl.BlockSpec((1, _D), lambda i: (0, 0)),
        ],
        out_specs=pl.BlockSpec((_BR, _D), lambda i: (i, 0)),
        out_shape=jax.ShapeDtypeStruct((_M, _D), jnp.float32),
    )(y2, st2, g2, b2)


def _pool_body(h_ref, batch_ref, pool_ref):
    @pl.when(pl.program_id(0) == 0)
    def _():
        pool_ref[...] = jnp.zeros_like(pool_ref)

    b = batch_ref[0, 0, :]
    oh = (b[None, :] == lax.broadcasted_iota(jnp.int32, (_G, _BR), 0)
          ).astype(jnp.float32)
    pool_ref[...] += jnp.dot(oh, h_ref[...],
                             preferred_element_type=jnp.float32)


def _pool(h, batch3):
    return pl.pallas_call(
        _pool_body,
        grid=(_NBLK,),
        in_specs=[
            pl.BlockSpec((_BR, _D), lambda i: (i, 0)),
            pl.BlockSpec((1, 1, _BR), lambda i: (i % _NXB, 0, 0)),
        ],
        out_specs=pl.BlockSpec((_G, _D), lambda i: (0, 0)),
        out_shape=jax.ShapeDtypeStruct((_G, _D), jnp.float32),
    )(h, batch3)


def _head_body(pools_ref, fcw_ref, fcb_ref, out_ref):
    acc = jnp.zeros((_G, _C), jnp.float32)
    for i in range(_LAYERS + 1):
        acc += jnp.dot(pools_ref[i] * (1.0 / _R), fcw_ref[i],
                       preferred_element_type=jnp.float32)
    acc += jnp.sum(fcb_ref[...], axis=0, keepdims=True)
    mx = jnp.max(acc, axis=-1, keepdims=True)
    sh = acc - mx
    out_ref[...] = sh - jnp.log(jnp.sum(jnp.exp(sh), axis=-1, keepdims=True))


def _head(pools, fc_w, fc_b):
    return pl.pallas_call(
        _head_body,
        out_shape=jax.ShapeDtypeStruct((_G, _C), jnp.float32),
    )(pools, fc_w, fc_b)


def kernel(x, edge_index, batch, conv_w1, conv_b1, conv_bng, conv_bnb,
           conv_w2, conv_b2, bn_g, bn_b, fc_w, fc_b):
    del conv_b1, conv_b2

    drop = jax.random.bernoulli(jax.random.key(42), _P, (_R, _N))
    keep = jnp.where(drop, 0.0, 1.0).astype(jnp.float32).reshape(_M, 1)

    offset = (jnp.max(edge_index) + 1).astype(jnp.int32)
    src, dst = edge_index[0], edge_index[1]
    perm = jnp.argsort(dst)
    dst_s = dst[perm]
    src_s = src[perm]
    roff = offset * jnp.arange(_R, dtype=jnp.int32)
    src_flat = (src_s[None, :] + roff[:, None]).reshape(_R * _E)
    dst_flat = (dst_s[None, :] + roff[:, None]).reshape(_R * _E)
    npad = 5120 * _BLK - _R * _E
    pad = jnp.full((npad,), 0, jnp.int32)
    padd = jnp.full((npad,), jnp.int32(1 << 29), jnp.int32)
    src_flat = jnp.concatenate([src_flat, pad])
    dst_flat = jnp.concatenate([dst_flat, padd])
    starts = jnp.arange(_NRANGE + 1, dtype=jnp.int32) * _RPT
    bounds = jnp.searchsorted(dst_flat[:_R * _E], starts).astype(jnp.int32)
    bounds = jnp.concatenate(
        [bounds, jnp.zeros((7,), jnp.int32)])

    batch3 = batch.reshape(_NXB, 1, _BR)

    h = _h0(x, keep)
    pools = [None] * (_LAYERS + 1)
    for i in range(_LAYERS):
        agg = _agg(h, src_flat, dst_flat.reshape(5120, _BLK),
                   bounds).reshape(_M, _D)
        y1, st1, pools[i] = _phase_a(h, agg, conv_w1[i], batch3)
        y2, st2 = _phase_b(y1, st1, conv_w2[i], conv_bng[i][None, :],
                           conv_bnb[i][None, :])
        h = _phase_c(y2, st2, bn_g[i][None, :], bn_b[i][None, :])
    pools[_LAYERS] = _pool(h, batch3)

    return _head(jnp.stack(pools), fc_w, fc_b)


# speedup vs baseline: 1.8094x; 1.7589x over previous
import functools

import jax
import jax.numpy as jnp
from jax import lax
from jax.experimental import pallas as pl
from jax.experimental.pallas import tpu as pltpu
from jax.experimental.pallas import tpu_sc as plsc

_N = 10000
_E = 160000
_F = 128
_D = 128
_C = 10
_R = 4
_G = 64
_P = 0.1
_LAYERS = 4
_M = _R * _N
_EPS = 1e-5

_NW = 32
_NRANGE = 64
_RPT = _M // _NRANGE
_BLK = 128
_GRP = _BLK // 16

_BR = 2000
_NBLK = _M // _BR
_NXB = _N // _BR


def _make_agg_kernel():
    mesh = plsc.VectorSubcoreMesh(core_axis_name="c", subcore_axis_name="s",
                                  num_cores=2, num_subcores=16)
    nc = 2

    @functools.partial(
        pl.kernel,
        out_type=jax.ShapeDtypeStruct((_NRANGE, _RPT, _D), jnp.float32),
        mesh=mesh,
        compiler_params=pltpu.CompilerParams(needs_layout_passes=False),
        scratch_types=[
            pltpu.VMEM((_BLK,), jnp.int32),
            pltpu.VMEM((_BLK,), jnp.int32),
            pltpu.VMEM((_BLK, _D), jnp.float32),
            pltpu.VMEM((_BLK, _D), jnp.float32),
            pltpu.VMEM((_RPT, _D), jnp.float32),
            pltpu.SMEM((_BLK,), jnp.int32),
            pltpu.SMEM((_NRANGE + 8,), jnp.int32),
            pltpu.VMEM_SHARED((_NRANGE + 8,), jnp.int32),
            pltpu.VMEM_SHARED((32, _BLK), jnp.int32),
            pltpu.SemaphoreType.DMA,
            pltpu.SemaphoreType.DMA,
            pltpu.SemaphoreType.DMA,
            pltpu.SemaphoreType.DMA,
            pltpu.SemaphoreType.DMA,
            pltpu.SemaphoreType.DMA,
        ],
    )
    def agg_kernel(h_hbm, src_hbm, dst_hbm, bounds_hbm, agg_hbm,
                   idx_a, idx_b, rows_a, rows_b, acc_v, dst_s, bnd_s,
                   bnd_sh, spd, sem_ia, sem_ib, sem_ra, sem_rb,
                   sem_da, sem_db):
        cid = lax.axis_index("c")
        sid = lax.axis_index("s")
        wid = sid * nc + cid
        slot_a = sid * 2
        slot_b = sid * 2 + 1

        @pl.when(sid == 0)
        def _():
            pltpu.sync_copy(bounds_hbm, bnd_sh)

        plsc.subcore_barrier()
        pltpu.sync_copy(bnd_sh, bnd_s)

        zero16 = jnp.zeros((16,), jnp.float32)
        nq = 4
        qsz = _BLK // nq

        def gather_issue(idx_v, rows_v, sem_r):
            for k in range(nq):
                pltpu.async_copy(h_hbm.at[idx_v.at[pl.ds(k * qsz, qsz)]],
                                 rows_v.at[pl.ds(k * qsz, qsz), :], sem_r)

        def gather_wait(idx_v, rows_v, sem_r):
            for k in range(nq):
                pltpu.make_async_copy(
                    h_hbm.at[idx_v.at[pl.ds(k * qsz, qsz)]],
                    rows_v.at[pl.ds(k * qsz, qsz), :], sem_r).wait()

        def stage_dst(base, slot, sem_d):
            drow = lax.shift_right_logical(base, 7)
            pltpu.make_async_copy(dst_hbm.at[drow], spd.at[slot],
                                  sem_d).wait()
            pltpu.sync_copy(spd.at[slot], dst_s)

        def edge_loop(base, rows_v, lo, hi, row_base):
            jlo = jnp.where(lo > base, lo - base, 0)
            jhi = jnp.where(hi - base < _BLK, hi - base, _BLK)

            @pl.loop(jlo, jhi)
            def _edge(j):
                r = dst_s[j] - row_base
                vals = [rows_v[j, pl.ds(c * 16, 16)]
                        for c in range(_D // 16)]
                for c in range(_D // 16):
                    plsc.addupdate(acc_v.at[r, pl.ds(c * 16, 16)], vals[c])

        @pl.loop(0, 2)
        def _range(qi):
            q = wid * 2 + qi
            row_base = q * _RPT
            lo = bnd_s[q]
            hi = bnd_s[q + 1]

            @pl.loop(0, _RPT * (_D // 16))
            def _zero(i):
                r = lax.shift_right_logical(i, 3)
                c = lax.shift_left(jnp.bitwise_and(i, 7), 4)
                acc_v[r, pl.ds(c, 16)] = zero16

            lo_al = lax.shift_left(lax.shift_right_logical(lo, 7), 7)
            nblk = lax.shift_right_logical(hi - lo_al + _BLK - 1, 7)

            def src_copy(blkno, idx_v, sem, slot, sem_d):
                base = pl.multiple_of(lo_al + lax.shift_left(blkno, 7), _BLK)
                pltpu.async_copy(dst_hbm.at[lax.shift_right_logical(base, 7)],
                                 spd.at[slot], sem_d)
                return pltpu.async_copy(src_hbm.at[pl.ds(base, _BLK)],
                                        idx_v, sem)

            @pl.when(nblk > 0)
            def _():
                src_copy(0, idx_a, sem_ia, slot_a, sem_da).wait()
                gather_issue(idx_a, rows_a, sem_ra)

                @pl.when(nblk > 1)
                def _():
                    src_copy(1, idx_b, sem_ib, slot_b, sem_db)

            @pl.when(nblk > 1)
            def _():
                pltpu.make_async_copy(
                    src_hbm.at[pl.ds(pl.multiple_of(lo_al + _BLK, _BLK),
                                     _BLK)],
                    idx_b, sem_ib).wait()
                gather_issue(idx_b, rows_b, sem_rb)

            def half_step(blk, idx_v, sem_i, rows_v, sem_r, slot, sem_d,
                          base):
                gather_wait(idx_v, rows_v, sem_r)
                stage_dst(base, slot, sem_d)

                @pl.when(blk + 2 < nblk)
                def _():
                    src_copy(blk + 2, idx_v, sem_i, slot, sem_d)

                edge_loop(base, rows_v, lo, hi, row_base)

                @pl.when(blk + 2 < nblk)
                def _():
                    pltpu.make_async_copy(
                        src_hbm.at[pl.ds(pl.multiple_of(base + 2 * _BLK,
                                                        _BLK), _BLK)],
                        idx_v, sem_i).wait()
                    gather_issue(idx_v, rows_v, sem_r)

            @pl.loop(0, lax.shift_right_logical(nblk + 1, 1))
            def _pair(i):
                a = lax.shift_left(i, 1)
                b = a + 1
                basea = pl.multiple_of(lo_al + lax.shift_left(a, 7), _BLK)
                baseb = basea + _BLK

                half_step(a, idx_a, sem_ia, rows_a, sem_ra, slot_a, sem_da,
                          basea)

                @pl.when(b < nblk)
                def _():
                    half_step(b, idx_b, sem_ib, rows_b, sem_rb, slot_b,
                              sem_db, baseb)

            pltpu.sync_copy(acc_v, agg_hbm.at[q])

    return agg_kernel


_AGG_CACHE = []


def _agg(h, src_flat, dst_flat, bounds):
    if not _AGG_CACHE:
        _AGG_CACHE.append(_make_agg_kernel())
    return _AGG_CACHE[0](h, src_flat, dst_flat, bounds)


def _h0_body(x_ref, keep_ref, h0_ref):
    h0_ref[...] = x_ref[...] * keep_ref[...]


def _h0(x, keep):
    return pl.pallas_call(
        _h0_body,
        grid=(_NBLK,),
        in_specs=[
            pl.BlockSpec((_BR, _F), lambda i: (i % _NXB, 0)),
            pl.BlockSpec((_BR, 1), lambda i: (i, 0)),
        ],
        out_specs=pl.BlockSpec((_BR, _F), lambda i: (i, 0)),
        out_shape=jax.ShapeDtypeStruct((_M, _F), jnp.float32),
    )(x, keep)


def _phase_a_body(h_ref, agg_ref, w1_ref, batch_ref, y1_ref, st_ref, pool_ref):
    @pl.when(pl.program_id(0) == 0)
    def _():
        st_ref[...] = jnp.zeros_like(st_ref)
        pool_ref[...] = jnp.zeros_like(pool_ref)

    h = h_ref[...]
    z = h + agg_ref[...]
    y1 = jnp.dot(z, w1_ref[...], preferred_element_type=jnp.float32)
    y1_ref[...] = y1
    s0 = jnp.sum(y1, axis=0, keepdims=True)
    s1 = jnp.sum(y1 * y1, axis=0, keepdims=True)
    st_ref[...] += jnp.concatenate(
        [s0, s1, jnp.zeros((6, _D), jnp.float32)], axis=0)
    b = batch_ref[0, 0, :]
    oh = (b[None, :] == lax.broadcasted_iota(jnp.int32, (_G, _BR), 0)
          ).astype(jnp.float32)
    pool_ref[...] += jnp.dot(oh, h, preferred_element_type=jnp.float32)


def _phase_a(h, agg, w1, batch3):
    return pl.pallas_call(
        _phase_a_body,
        grid=(_NBLK,),
        in_specs=[
            pl.BlockSpec((_BR, _D), lambda i: (i, 0)),
            pl.BlockSpec((_BR, _D), lambda i: (i, 0)),
            pl.BlockSpec((_D, _D), lambda i: (0, 0)),
            pl.BlockSpec((1, 1, _BR), lambda i: (i % _NXB, 0, 0)),
        ],
        out_specs=[
            pl.BlockSpec((_BR, _D), lambda i: (i, 0)),
            pl.BlockSpec((8, _D), lambda i: (0, 0)),
            pl.BlockSpec((_G, _D), lambda i: (0, 0)),
        ],
        out_shape=[
            jax.ShapeDtypeStruct((_M, _D), jnp.float32),
            jax.ShapeDtypeStruct((8, _D), jnp.float32),
            jax.ShapeDtypeStruct((_G, _D), jnp.float32),
        ],
    )(h, agg, w1, batch3)


def _bn_from_stats(y, st_ref, g, b):
    mean = st_ref[0:1, :] * (1.0 / _M)
    var = st_ref[1:2, :] * (1.0 / _M) - mean * mean
    inv = lax.rsqrt(var + _EPS)
    return (y - mean) * inv * g + b


def _phase_b_body(y1_ref, st_ref, w2_ref, g1_ref, b1_ref, y2_ref, st2_ref):
    @pl.when(pl.program_id(0) == 0)
    def _():
        st2_ref[...] = jnp.zeros_like(st2_ref)

    y1n = _bn_from_stats(y1_ref[...], st_ref, g1_ref[...], b1_ref[...])
    y1n = jnp.maximum(y1n, 0.0)
    y2 = jnp.dot(y1n, w2_ref[...], preferred_element_type=jnp.float32)
    y2_ref[...] = y2
    s0 = jnp.sum(y2, axis=0, keepdims=True)
    s1 = jnp.sum(y2 * y2, axis=0, keepdims=True)
    st2_ref[...] += jnp.concatenate(
        [s0, s1, jnp.zeros((6, _D), jnp.float32)], axis=0)


def _phase_b(y1, st1, w2, g1, b1):
    return pl.pallas_call(
        _phase_b_body,
        grid=(_NBLK,),
        in_specs=[
            pl.BlockSpec((_BR, _D), lambda i: (i, 0)),
            pl.BlockSpec((8, _D), lambda i: (0, 0)),
            pl.BlockSpec((_D, _D), lambda i: (0, 0)),
            pl.BlockSpec((1, _D), lambda i: (0, 0)),
            pl.BlockSpec((1, _D), lambda i: (0, 0)),
        ],
        out_specs=[
            pl.BlockSpec((_BR, _D), lambda i: (i, 0)),
            pl.BlockSpec((8, _D), lambda i: (0, 0)),
        ],
        out_shape=[
            jax.ShapeDtypeStruct((_M, _D), jnp.float32),
            jax.ShapeDtypeStruct((8, _D), jnp.float32),
        ],
    )(y1, st1, w2, g1, b1)


def _phase_c_body(y2_ref, st_ref, g2_ref, b2_ref, h_ref):
    h = _bn_from_stats(y2_ref[...], st_ref, g2_ref[...], b2_ref[...])
    h_ref[...] = jnp.maximum(h, 0.0)


def _phase_c(y2, st2, g2, b2):
    return pl.pallas_call(
        _phase_c_body,
        grid=(_NBLK,),
        in_specs=[
            pl.BlockSpec((_BR, _D), lambda i: (i, 0)),
            pl.BlockSpec((8, _D), lambda i: (0, 0)),
            pl.BlockSpec((1, _D), lambda i: (0, 0)),
            pl.BlockSpec((1, _D), lambda i: (0, 0)),
        ],
        out_specs=pl.BlockSpec((_BR, _D), lambda i: (i, 0)),
        out_shape=jax.ShapeDtypeStruct((_M, _D), jnp.float32),
    )(y2, st2, g2, b2)


def _pool_body(h_ref, batch_ref, pool_ref):
    @pl.when(pl.program_id(0) == 0)
    def _():
        pool_ref[...] = jnp.zeros_like(pool_ref)

    b = batch_ref[0, 0, :]
    oh = (b[None, :] == lax.broadcasted_iota(jnp.int32, (_G, _BR), 0)
          ).astype(jnp.float32)
    pool_ref[...] += jnp.dot(oh, h_ref[...],
                             preferred_element_type=jnp.float32)


def _pool(h, batch3):
    return pl.pallas_call(
        _pool_body,
        grid=(_NBLK,),
        in_specs=[
            pl.BlockSpec((_BR, _D), lambda i: (i, 0)),
            pl.BlockSpec((1, 1, _BR), lambda i: (i % _NXB, 0, 0)),
        ],
        out_specs=pl.BlockSpec((_G, _D), lambda i: (0, 0)),
        out_shape=jax.ShapeDtypeStruct((_G, _D), jnp.float32),
    )(h, batch3)


def _head_body(pools_ref, fcw_ref, fcb_ref, out_ref):
    acc = jnp.zeros((_G, _C), jnp.float32)
    for i in range(_LAYERS + 1):
        acc += jnp.dot(pools_ref[i] * (1.0 / _R), fcw_ref[i],
                       preferred_element_type=jnp.float32)
    acc += jnp.sum(fcb_ref[...], axis=0, keepdims=True)
    mx = jnp.max(acc, axis=-1, keepdims=True)
    sh = acc - mx
    out_ref[...] = sh - jnp.log(jnp.sum(jnp.exp(sh), axis=-1, keepdims=True))


def _head(pools, fc_w, fc_b):
    return pl.pallas_call(
        _head_body,
        out_shape=jax.ShapeDtypeStruct((_G, _C), jnp.float32),
    )(pools, fc_w, fc_b)


def kernel(x, edge_index, batch, conv_w1, conv_b1, conv_bng, conv_bnb,
           conv_w2, conv_b2, bn_g, bn_b, fc_w, fc_b):
    del conv_b1, conv_b2

    drop = jax.random.bernoulli(jax.random.key(42), _P, (_R, _N))
    keep = jnp.where(drop, 0.0, 1.0).astype(jnp.float32).reshape(_M, 1)

    offset = (jnp.max(edge_index) + 1).astype(jnp.int32)
    src, dst = edge_index[0], edge_index[1]
    perm = jnp.argsort(dst)
    dst_s = dst[perm]
    src_s = src[perm]
    roff = offset * jnp.arange(_R, dtype=jnp.int32)
    src_flat = (src_s[None, :] + roff[:, None]).reshape(_R * _E)
    dst_flat = (dst_s[None, :] + roff[:, None]).reshape(_R * _E)
    npad = 5120 * _BLK - _R * _E
    pad = jnp.full((npad,), 0, jnp.int32)
    padd = jnp.full((npad,), jnp.int32(1 << 29), jnp.int32)
    src_flat = jnp.concatenate([src_flat, pad])
    dst_flat = jnp.concatenate([dst_flat, padd])
    starts = jnp.arange(_NRANGE + 1, dtype=jnp.int32) * _RPT
    bounds = jnp.searchsorted(dst_flat[:_R * _E], starts).astype(jnp.int32)
    bounds = jnp.concatenate(
        [bounds, jnp.zeros((7,), jnp.int32)])

    batch3 = batch.reshape(_NXB, 1, _BR)

    h = _h0(x, keep)
    pools = [None] * (_LAYERS + 1)
    for i in range(_LAYERS):
        agg = _agg(h, src_flat, dst_flat.reshape(5120, _BLK),
                   bounds).reshape(_M, _D)
        y1, st1, pools[i] = _phase_a(h, agg, conv_w1[i], batch3)
        y2, st2 = _phase_b(y1, st1, conv_w2[i], conv_bng[i][None, :],
                           conv_bnb[i][None, :])
        h = _phase_c(y2, st2, bn_g[i][None, :], bn_b[i][None, :])
    pools[_LAYERS] = _pool(h, batch3)

    return _head(jnp.stack(pools), fc_w, fc_b)
